# Initial kernel scaffold; baseline (speedup 1.0000x reference)
import jax
import jax.numpy as jnp
from jax.experimental import pallas as pl


def kernel(x, edge_index, edge_weights, W1, b1, W2, b2):
    raise NotImplementedError("write your pallas kernel here")


# speedup vs baseline: 18.5761x; 18.5761x over previous
import functools

import jax
import jax.numpy as jnp
from jax import lax
from jax.experimental import pallas as pl
from jax.experimental.pallas import tpu as pltpu
from jax.experimental.pallas import tpu_sc as plsc

N = 10000
E = 320000
D = 128

NC = 2
NS = 16
NW = NC * NS
LANES = 16

CH = 128
NCHUNK = 80
TPE = NCHUNK * CH
EPAD = NW * TPE
NPAD = 10240
RPT = NPAD // NS
BLK = 512

_mesh = plsc.VectorSubcoreMesh(core_axis_name="c", subcore_axis_name="s")


@functools.partial(
    pl.kernel,
    mesh=_mesh,
    out_type=jax.ShapeDtypeStruct((NC, NPAD), jnp.float32),
    compiler_params=pltpu.CompilerParams(needs_layout_passes=False),
    scratch_types=[
        pltpu.VMEM_SHARED((NPAD,), jnp.float32),
        pltpu.VMEM((NCHUNK, CH), jnp.int32),
        pltpu.VMEM((NCHUNK, CH), jnp.float32),
        pltpu.VMEM((RPT,), jnp.float32),
    ],
)
def _deg_kernel(col_hbm, ew_hbm, deg_out, dacc, cbuf, wbuf, zbuf):
    cid = lax.axis_index("c")
    sid = lax.axis_index("s")
    wid = cid * NS + sid
    pltpu.sync_copy(col_hbm.at[wid], cbuf)
    pltpu.sync_copy(ew_hbm.at[wid], wbuf)
    zeros16 = jnp.zeros((LANES,), jnp.float32)

    def _zero(i, carry):
        zbuf[pl.ds(i * LANES, LANES)] = zeros16
        return carry

    lax.fori_loop(0, RPT // LANES, _zero, 0)
    pltpu.sync_copy(zbuf, dacc.at[pl.ds(sid * RPT, RPT)])
    plsc.subcore_barrier()

    def _scat(j, carry):
        pltpu.sync_copy(wbuf.at[j], dacc.at[cbuf.at[j]], add=True)
        return carry

    lax.fori_loop(0, NCHUNK, _scat, 0)
    plsc.subcore_barrier()
    pltpu.sync_copy(dacc.at[pl.ds(sid * RPT, RPT)],
                    deg_out.at[cid, pl.ds(sid * RPT, RPT)])


@functools.partial(
    pl.kernel,
    mesh=_mesh,
    out_type=jax.ShapeDtypeStruct((NC, NPAD, D), jnp.float32),
    compiler_params=pltpu.CompilerParams(needs_layout_passes=False),
    scratch_types=[
        pltpu.VMEM_SHARED((NPAD, D), jnp.float32),
        pltpu.VMEM((3, CH), jnp.int32),
        pltpu.VMEM((3, CH), jnp.int32),
        pltpu.VMEM((CH, D), jnp.float32),
        pltpu.VMEM((CH, D), jnp.float32),
        pltpu.SemaphoreType.DMA,
        pltpu.SemaphoreType.DMA,
    ],
)
def _agg_kernel(g_hbm, edge_hbm, agg_out,
                acc, ebuf0, ebuf1, buf0, buf1, gsem0, gsem1):
    cid = lax.axis_index("c")
    sid = lax.axis_index("s")
    wid = cid * NS + sid

    zeros16 = jnp.zeros((LANES,), jnp.float32)

    def _zero(r, carry):
        for f in range(D // LANES):
            buf0[r, pl.ds(f * LANES, LANES)] = zeros16
        return carry

    lax.fori_loop(0, CH, _zero, 0)
    for t in range(RPT // CH):
        pltpu.sync_copy(buf0, acc.at[pl.ds(sid * RPT + t * CH, CH)])
    plsc.subcore_barrier()

    def _edges_load(j, ebuf):
        pltpu.sync_copy(edge_hbm.at[wid, j], ebuf)

    def _gather_start(ebuf, buf, sem):
        pltpu.async_copy(g_hbm.at[ebuf.at[0]], buf, sem)

    def _gather_wait(ebuf, buf, sem):
        pltpu.make_async_copy(g_hbm.at[ebuf.at[0]], buf, sem).wait()

    def _scale(ebuf, buf):
        def _edge(e, carry):
            w16i = plsc.load_gather(
                ebuf, [jnp.full((LANES,), 2, jnp.int32),
                       jnp.full((LANES,), e, jnp.int32)])
            w16 = plsc.bitcast(w16i, jnp.float32)
            for f in range(D // LANES):
                sl = pl.ds(f * LANES, LANES)
                buf[e, sl] = buf[e, sl] * w16
            return carry

        lax.fori_loop(0, CH, _edge, 0)

    def _scatter(ebuf, buf):
        pltpu.sync_copy(buf, acc.at[ebuf.at[1]], add=True)

    _edges_load(0, ebuf0)
    _gather_start(ebuf0, buf0, gsem0)

    def _pair(t, carry):
        j0 = t * 2
        _edges_load(j0 + 1, ebuf1)
        _gather_start(ebuf1, buf1, gsem1)
        _gather_wait(ebuf0, buf0, gsem0)
        _scale(ebuf0, buf0)
        _scatter(ebuf0, buf0)

        @pl.when(t < NCHUNK // 2 - 1)
        def _():
            _edges_load(j0 + 2, ebuf0)
            _gather_start(ebuf0, buf0, gsem0)

        _gather_wait(ebuf1, buf1, gsem1)
        _scale(ebuf1, buf1)
        _scatter(ebuf1, buf1)
        return carry

    lax.fori_loop(0, NCHUNK // 2, _pair, 0)
    plsc.subcore_barrier()
    pltpu.sync_copy(acc.at[pl.ds(sid * RPT, RPT)],
                    agg_out.at[cid, pl.ds(sid * RPT, RPT)])


def _mm1_body(x_ref, w_ref, d0_ref, d1_ref, g_ref, dv_ref):
    deg = d0_ref[...] + d1_ref[...] + 1.0
    dv = jnp.where(deg > 0, lax.rsqrt(jnp.where(deg > 0, deg, 1.0)), 0.0)
    h = jnp.dot(x_ref[...], w_ref[...], preferred_element_type=jnp.float32)
    g_ref[...] = h * dv
    dv_ref[...] = dv


def _mm2_body(a0_ref, a1_ref, g_ref, dv_ref, b_ref, w_ref, g2_ref):
    dv = dv_ref[...]
    pre = dv * (a0_ref[...] + a1_ref[...] + g_ref[...]) + b_ref[...]
    h = jnp.maximum(pre, 0.0)
    g2_ref[...] = dv * jnp.dot(h, w_ref[...], preferred_element_type=jnp.float32)


def _fin_body(a0_ref, a1_ref, g_ref, dv_ref, b_ref, o_ref):
    dv = dv_ref[...]
    pre = dv * (a0_ref[...] + a1_ref[...] + g_ref[...]) + b_ref[...]
    o_ref[...] = jnp.maximum(pre, 0.0)


_row_spec = pl.BlockSpec((BLK, D), lambda i: (i, 0))
_col_spec = pl.BlockSpec((BLK, 1), lambda i: (i, 0))
_w_spec = pl.BlockSpec((D, D), lambda i: (0, 0))
_b_spec = pl.BlockSpec((1, D), lambda i: (0, 0))
_GRID = (NPAD // BLK,)


def _mm1(x_pad, w1t, d0, d1):
    return pl.pallas_call(
        _mm1_body,
        grid=_GRID,
        in_specs=[_row_spec, _w_spec, _col_spec, _col_spec],
        out_specs=[_row_spec, _col_spec],
        out_shape=[jax.ShapeDtypeStruct((NPAD, D), jnp.float32),
                   jax.ShapeDtypeStruct((NPAD, 1), jnp.float32)],
    )(x_pad, w1t, d0, d1)


def _mm2(a0, a1, g, dv, b, w2t):
    return pl.pallas_call(
        _mm2_body,
        grid=_GRID,
        in_specs=[_row_spec, _row_spec, _row_spec, _col_spec, _b_spec, _w_spec],
        out_specs=[_row_spec],
        out_shape=[jax.ShapeDtypeStruct((NPAD, D), jnp.float32)],
    )(a0, a1, g, dv, b, w2t)[0]


def _fin(a0, a1, g, dv, b):
    return pl.pallas_call(
        _fin_body,
        grid=_GRID,
        in_specs=[_row_spec, _row_spec, _row_spec, _col_spec, _b_spec],
        out_specs=[_row_spec],
        out_shape=[jax.ShapeDtypeStruct((NPAD, D), jnp.float32)],
    )(a0, a1, g, dv, b)[0]


def kernel(x, edge_index, edge_weights, W1, b1, W2, b2):
    f32, i32 = jnp.float32, jnp.int32
    row = edge_index[0]
    col = edge_index[1]
    pad = EPAD - E
    ar = jnp.arange(pad, dtype=i32)
    row_p = jnp.concatenate([row, ar % N])
    col_p = jnp.concatenate([col, N + ar % (NPAD - N)])
    ew_p = jnp.concatenate([edge_weights.astype(f32), jnp.zeros((pad,), f32)])
    row3 = row_p.reshape(NW, NCHUNK, CH)
    col3 = col_p.reshape(NW, NCHUNK, CH)
    ew3 = ew_p.reshape(NW, NCHUNK, CH)
    ewb3 = lax.bitcast_convert_type(ew3, i32)
    earr = jnp.stack([row3, col3, ewb3], axis=2)
    x_pad = jnp.pad(x.astype(f32), ((0, NPAD - N), (0, 0)))
    w1t = W1.astype(f32).T
    w2t = W2.astype(f32).T
    b1r = b1.astype(f32).reshape(1, D)
    b2r = b2.astype(f32).reshape(1, D)

    deg_parts = _deg_kernel(col3, ew3)
    d0 = deg_parts[0].reshape(NPAD, 1)
    d1 = deg_parts[1].reshape(NPAD, 1)

    g1, dv = _mm1(x_pad, w1t, d0, d1)
    agg1 = _agg_kernel(g1, earr)
    g2 = _mm2(agg1[0], agg1[1], g1, dv, b1r, w2t)
    agg2 = _agg_kernel(g2, earr)
    out = _fin(agg2[0], agg2[1], g2, dv, b2r)
    return out[:N]


# speedup vs baseline: 19.8059x; 1.0662x over previous
import functools

import jax
import jax.numpy as jnp
from jax import lax
from jax.experimental import pallas as pl
from jax.experimental.pallas import tpu as pltpu
from jax.experimental.pallas import tpu_sc as plsc

N = 10000
E = 320000
D = 128

NC = 2
NS = 16
NW = NC * NS
LANES = 16

CH = 128
NCHUNK = 80
TPE = NCHUNK * CH
EPAD = NW * TPE
NPAD = 10240
RPT = NPAD // NS
BLK = 512

_mesh = plsc.VectorSubcoreMesh(core_axis_name="c", subcore_axis_name="s")


@functools.partial(
    pl.kernel,
    mesh=_mesh,
    out_type=jax.ShapeDtypeStruct((NC, NPAD), jnp.float32),
    compiler_params=pltpu.CompilerParams(needs_layout_passes=False),
    scratch_types=[
        pltpu.VMEM_SHARED((NPAD,), jnp.float32),
        pltpu.VMEM((NCHUNK, CH), jnp.int32),
        pltpu.VMEM((NCHUNK, CH), jnp.float32),
        pltpu.VMEM((RPT,), jnp.float32),
    ],
)
def _deg_kernel(col_hbm, ew_hbm, deg_out, dacc, cbuf, wbuf, zbuf):
    cid = lax.axis_index("c")
    sid = lax.axis_index("s")
    wid = cid * NS + sid
    pltpu.sync_copy(col_hbm.at[wid], cbuf)
    pltpu.sync_copy(ew_hbm.at[wid], wbuf)
    zeros16 = jnp.zeros((LANES,), jnp.float32)

    def _zero(i, carry):
        zbuf[pl.ds(i * LANES, LANES)] = zeros16
        return carry

    lax.fori_loop(0, RPT // LANES, _zero, 0)
    pltpu.sync_copy(zbuf, dacc.at[pl.ds(sid * RPT, RPT)])
    plsc.subcore_barrier()

    def _scat(j, carry):
        pltpu.sync_copy(wbuf.at[j], dacc.at[cbuf.at[j]], add=True)
        return carry

    lax.fori_loop(0, NCHUNK, _scat, 0)
    plsc.subcore_barrier()
    pltpu.sync_copy(dacc.at[pl.ds(sid * RPT, RPT)],
                    deg_out.at[cid, pl.ds(sid * RPT, RPT)])


@functools.partial(
    pl.kernel,
    mesh=_mesh,
    out_type=jax.ShapeDtypeStruct((NC, NPAD, D), jnp.float32),
    compiler_params=pltpu.CompilerParams(needs_layout_passes=False),
    scratch_types=[
        pltpu.VMEM_SHARED((NPAD, D), jnp.float32),
        pltpu.VMEM((3, CH), jnp.int32),
        pltpu.VMEM((3, CH), jnp.int32),
        pltpu.VMEM((CH, D), jnp.float32),
        pltpu.VMEM((CH, D), jnp.float32),
        pltpu.SemaphoreType.DMA,
        pltpu.SemaphoreType.DMA,
        pltpu.SemaphoreType.DMA,
        pltpu.SemaphoreType.DMA,
    ],
)
def _agg_kernel(g_hbm, edge_hbm, agg_out,
                acc, ebuf0, ebuf1, buf0, buf1, gsem0, gsem1, ssem0, ssem1):
    cid = lax.axis_index("c")
    sid = lax.axis_index("s")
    wid = cid * NS + sid

    zeros16 = jnp.zeros((LANES,), jnp.float32)

    def _zero(r, carry):
        for f in range(D // LANES):
            buf0[r, pl.ds(f * LANES, LANES)] = zeros16
        return carry

    lax.fori_loop(0, CH, _zero, 0)
    for t in range(RPT // CH):
        pltpu.sync_copy(buf0, acc.at[pl.ds(sid * RPT + t * CH, CH)])
    plsc.subcore_barrier()

    def _edges_load(j, ebuf):
        pltpu.sync_copy(edge_hbm.at[wid, j], ebuf)

    def _gather_start(ebuf, buf, sem):
        pltpu.async_copy(g_hbm.at[ebuf.at[0]], buf, sem)

    def _gather_wait(ebuf, buf, sem):
        pltpu.make_async_copy(g_hbm.at[ebuf.at[0]], buf, sem).wait()

    def _scale(ebuf, buf):
        def _edge(e, carry):
            w16i = plsc.load_gather(
                ebuf, [jnp.full((LANES,), 2, jnp.int32),
                       jnp.full((LANES,), e, jnp.int32)])
            w16 = plsc.bitcast(w16i, jnp.float32)
            for f in range(D // LANES):
                sl = pl.ds(f * LANES, LANES)
                buf[e, sl] = buf[e, sl] * w16
            return carry

        lax.fori_loop(0, CH, _edge, 0)

    def _scatter_start(ebuf, buf, sem):
        pltpu.async_copy(buf, acc.at[ebuf.at[1]], sem, add=True)

    def _scatter_wait(ebuf, buf, sem):
        pltpu.make_async_copy(buf, acc.at[ebuf.at[1]], sem).wait()

    _edges_load(0, ebuf0)
    _gather_start(ebuf0, buf0, gsem0)
    _edges_load(1, ebuf1)
    _gather_start(ebuf1, buf1, gsem1)

    def _pair(t, carry):
        j0 = t * 2
        _gather_wait(ebuf0, buf0, gsem0)
        _scale(ebuf0, buf0)
        _scatter_start(ebuf0, buf0, ssem0)
        _gather_wait(ebuf1, buf1, gsem1)
        _scale(ebuf1, buf1)
        _scatter_start(ebuf1, buf1, ssem1)

        @pl.when(t < NCHUNK // 2 - 1)
        def _():
            _scatter_wait(ebuf0, buf0, ssem0)
            _edges_load(j0 + 2, ebuf0)
            _gather_start(ebuf0, buf0, gsem0)
            _scatter_wait(ebuf1, buf1, ssem1)
            _edges_load(j0 + 3, ebuf1)
            _gather_start(ebuf1, buf1, gsem1)

        return carry

    lax.fori_loop(0, NCHUNK // 2, _pair, 0)
    _scatter_wait(ebuf0, buf0, ssem0)
    _scatter_wait(ebuf1, buf1, ssem1)
    plsc.subcore_barrier()
    pltpu.sync_copy(acc.at[pl.ds(sid * RPT, RPT)],
                    agg_out.at[cid, pl.ds(sid * RPT, RPT)])


def _mm1_body(x_ref, w_ref, d0_ref, d1_ref, g_ref, dv_ref):
    deg = d0_ref[...] + d1_ref[...] + 1.0
    dv = jnp.where(deg > 0, lax.rsqrt(jnp.where(deg > 0, deg, 1.0)), 0.0)
    h = jnp.dot(x_ref[...], w_ref[...], preferred_element_type=jnp.float32)
    g_ref[...] = h * dv
    dv_ref[...] = dv


def _mm2_body(a0_ref, a1_ref, g_ref, dv_ref, b_ref, w_ref, g2_ref):
    dv = dv_ref[...]
    pre = dv * (a0_ref[...] + a1_ref[...] + g_ref[...]) + b_ref[...]
    h = jnp.maximum(pre, 0.0)
    g2_ref[...] = dv * jnp.dot(h, w_ref[...], preferred_element_type=jnp.float32)


def _fin_body(a0_ref, a1_ref, g_ref, dv_ref, b_ref, o_ref):
    dv = dv_ref[...]
    pre = dv * (a0_ref[...] + a1_ref[...] + g_ref[...]) + b_ref[...]
    o_ref[...] = jnp.maximum(pre, 0.0)


_row_spec = pl.BlockSpec((BLK, D), lambda i: (i, 0))
_col_spec = pl.BlockSpec((BLK, 1), lambda i: (i, 0))
_w_spec = pl.BlockSpec((D, D), lambda i: (0, 0))
_b_spec = pl.BlockSpec((1, D), lambda i: (0, 0))
_GRID = (NPAD // BLK,)


def _mm1(x_pad, w1t, d0, d1):
    return pl.pallas_call(
        _mm1_body,
        grid=_GRID,
        in_specs=[_row_spec, _w_spec, _col_spec, _col_spec],
        out_specs=[_row_spec, _col_spec],
        out_shape=[jax.ShapeDtypeStruct((NPAD, D), jnp.float32),
                   jax.ShapeDtypeStruct((NPAD, 1), jnp.float32)],
    )(x_pad, w1t, d0, d1)


def _mm2(a0, a1, g, dv, b, w2t):
    return pl.pallas_call(
        _mm2_body,
        grid=_GRID,
        in_specs=[_row_spec, _row_spec, _row_spec, _col_spec, _b_spec, _w_spec],
        out_specs=[_row_spec],
        out_shape=[jax.ShapeDtypeStruct((NPAD, D), jnp.float32)],
    )(a0, a1, g, dv, b, w2t)[0]


def _fin(a0, a1, g, dv, b):
    return pl.pallas_call(
        _fin_body,
        grid=_GRID,
        in_specs=[_row_spec, _row_spec, _row_spec, _col_spec, _b_spec],
        out_specs=[_row_spec],
        out_shape=[jax.ShapeDtypeStruct((NPAD, D), jnp.float32)],
    )(a0, a1, g, dv, b)[0]


def kernel(x, edge_index, edge_weights, W1, b1, W2, b2):
    f32, i32 = jnp.float32, jnp.int32
    row = edge_index[0]
    col = edge_index[1]
    pad = EPAD - E
    ar = jnp.arange(pad, dtype=i32)
    row_p = jnp.concatenate([row, ar % N])
    col_p = jnp.concatenate([col, N + ar % (NPAD - N)])
    ew_p = jnp.concatenate([edge_weights.astype(f32), jnp.zeros((pad,), f32)])
    row3 = row_p.reshape(NW, NCHUNK, CH)
    col3 = col_p.reshape(NW, NCHUNK, CH)
    ew3 = ew_p.reshape(NW, NCHUNK, CH)
    ewb3 = lax.bitcast_convert_type(ew3, i32)
    earr = jnp.stack([row3, col3, ewb3], axis=2)
    x_pad = jnp.pad(x.astype(f32), ((0, NPAD - N), (0, 0)))
    w1t = W1.astype(f32).T
    w2t = W2.astype(f32).T
    b1r = b1.astype(f32).reshape(1, D)
    b2r = b2.astype(f32).reshape(1, D)

    deg_parts = _deg_kernel(col3, ew3)
    d0 = deg_parts[0].reshape(NPAD, 1)
    d1 = deg_parts[1].reshape(NPAD, 1)

    g1, dv = _mm1(x_pad, w1t, d0, d1)
    agg1 = _agg_kernel(g1, earr)
    g2 = _mm2(agg1[0], agg1[1], g1, dv, b1r, w2t)
    agg2 = _agg_kernel(g2, earr)
    out = _fin(agg2[0], agg2[1], g2, dv, b2r)
    return out[:N]
